# baseline (speedup 1.0000x reference)
import functools

import jax
import jax.numpy as jnp
from jax import lax
from jax.experimental import pallas as pl
from jax.experimental.pallas import tpu as pltpu
from jax.experimental.pallas import tpu_sc as plsc

B = 16384
D = 64
NUM_ROWS2 = 500000
NC = 2
NS = 16
L = 16
NW = NC * NS
BPW = B // NW
CHUNK = 128
NCHUNK = BPW // CHUNK
GPC = CHUNK // L


def _body(uid_hbm, iid_hbm, uemb_hbm, iemb_hbm, ubw_hbm, ibw_hbm, gb_hbm,
          out_hbm,
          uid_v, iid_v, urow_idx, irow_idx, urows_v, irows_v, ub_v, ib_v,
          gb_v, out_v, sem_u, sem_i, sem_ub, sem_ib):
    wid = lax.axis_index("s") * NC + lax.axis_index("c")
    base = wid * BPW

    bias_copies = []
    for c in range(NCHUNK):
        src = pl.ds(base + c * CHUNK, CHUNK)
        pltpu.sync_copy(uid_hbm.at[src], uid_v.at[c])
        pltpu.sync_copy(iid_hbm.at[src], iid_v.at[c])
    pltpu.sync_copy(gb_hbm, gb_v.at[pl.ds(0, 1)])
    for c in range(NCHUNK):
        for j in range(CHUNK // L):
            sl = pl.ds(j * L, L)
            uid = uid_v[c, sl]
            iid = iid_v[c, sl]
            urow_idx[c, sl] = lax.shift_right_logical(uid, 1)
            irow_idx[c, sl] = lax.shift_right_logical(iid, 1)
    for c in range(NCHUNK):
        rsl = pl.ds(c * CHUNK, CHUNK)
        bias_copies.append(pltpu.async_copy(ubw_hbm.at[uid_v.at[c]],
                                            ub_v.at[rsl], sem_ub))
        bias_copies.append(pltpu.async_copy(ibw_hbm.at[iid_v.at[c]],
                                            ib_v.at[rsl], sem_ib))

    gb = gb_v[pl.ds(0, L)][0]
    iota16 = lax.iota(jnp.int32, L)
    for cp in bias_copies:
        cp.wait()

    def chunk_step(c, carry):
        cp_u = pltpu.async_copy(uemb_hbm.at[urow_idx.at[c]], urows_v, sem_u)
        cp_i = pltpu.async_copy(iemb_hbm.at[irow_idx.at[c]], irows_v, sem_i)
        cp_u.wait()
        cp_i.wait()
        for g in range(GPC):
            rows = g * L + iota16
            gsl = pl.ds(g * L, L)
            up64 = (uid_v[c, gsl] & 1) * D
            ip64 = (iid_v[c, gsl] & 1) * D
            abs_rows = c * CHUNK + rows
            acc = plsc.load_gather(ub_v, [abs_rows])
            acc = acc + plsc.load_gather(ib_v, [abs_rows])
            acc = acc + gb
            for d in range(D):
                u = plsc.load_gather(urows_v, [rows, up64 + d])
                i = plsc.load_gather(irows_v, [rows, ip64 + d])
                acc = acc + u * i
            out_v[c, gsl] = acc
        return carry

    lax.fori_loop(0, NCHUNK, chunk_step, 0)

    for c in range(NCHUNK):
        pltpu.sync_copy(out_v.at[c], out_hbm.at[pl.ds(base + c * CHUNK, CHUNK)])


@jax.jit
def _mf_predict(user_ids, item_ids, user_emb2, item_emb2,
                user_bias_w, item_bias_w, global_bias):
    mesh = plsc.VectorSubcoreMesh(core_axis_name="c", subcore_axis_name="s",
                                  num_cores=NC, num_subcores=NS)
    kfn = pl.kernel(
        _body,
        out_type=jax.ShapeDtypeStruct((B,), jnp.float32),
        mesh=mesh,
        scratch_types=[
            pltpu.VMEM((NCHUNK, CHUNK), jnp.int32),
            pltpu.VMEM((NCHUNK, CHUNK), jnp.int32),
            pltpu.VMEM((NCHUNK, CHUNK), jnp.int32),
            pltpu.VMEM((NCHUNK, CHUNK), jnp.int32),
            pltpu.VMEM((CHUNK, 2 * D), jnp.float32),
            pltpu.VMEM((CHUNK, 2 * D), jnp.float32),
            pltpu.VMEM((BPW,), jnp.float32),
            pltpu.VMEM((BPW,), jnp.float32),
            pltpu.VMEM((L,), jnp.float32),
            pltpu.VMEM((NCHUNK, CHUNK), jnp.float32),
            pltpu.SemaphoreType.DMA,
            pltpu.SemaphoreType.DMA,
            pltpu.SemaphoreType.DMA,
            pltpu.SemaphoreType.DMA,
        ],
        compiler_params=pltpu.CompilerParams(needs_layout_passes=False,
                                             use_tc_tiling_on_sc=False),
    )
    return kfn(user_ids, item_ids, user_emb2, item_emb2,
               user_bias_w, item_bias_w, global_bias)


def kernel(user_ids, item_ids, user_emb, item_emb, user_bias_w, item_bias_w,
           global_bias):
    return _mf_predict(user_ids.astype(jnp.int32), item_ids.astype(jnp.int32),
                       user_emb.reshape(NUM_ROWS2, 2 * D),
                       item_emb.reshape(NUM_ROWS2, 2 * D),
                       user_bias_w.reshape(-1), item_bias_w.reshape(-1),
                       global_bias)


# speedup vs baseline: 1.9731x; 1.9731x over previous
import functools

import jax
import jax.numpy as jnp
from jax import lax
from jax.experimental import pallas as pl
from jax.experimental.pallas import tpu as pltpu
from jax.experimental.pallas import tpu_sc as plsc

B = 16384
D = 64
NC = 2
NS = 16
L = 16
NW = NC * NS
BPW = B // NW
CHUNK = 128
NCHUNK = BPW // CHUNK
G = 4
NWAVE = L // G
NSG = BPW // L


def _body(uid_hbm, iid_hbm, uemb_hbm, iemb_hbm, ubw_hbm, ibw_hbm, gb_hbm,
          out_hbm,
          uid_v, iid_v, ubufs, ibufs, ub_v, ib_v,
          gb_v, out_v, sem_u, sem_i, sem_ub, sem_ib):
    wid = lax.axis_index("s") * NC + lax.axis_index("c")
    base = wid * BPW

    for c in range(NCHUNK):
        src = pl.ds(base + c * CHUNK, CHUNK)
        pltpu.sync_copy(uid_hbm.at[src], uid_v.at[c])
        pltpu.sync_copy(iid_hbm.at[src], iid_v.at[c])
    pltpu.sync_copy(gb_hbm, gb_v.at[pl.ds(0, 1)])

    bias_copies = []
    for c in range(NCHUNK):
        rsl = pl.ds(c * CHUNK, CHUNK)
        bias_copies.append(pltpu.async_copy(ubw_hbm.at[uid_v.at[c]],
                                            ub_v.at[rsl], sem_ub))
        bias_copies.append(pltpu.async_copy(ibw_hbm.at[iid_v.at[c]],
                                            ib_v.at[rsl], sem_ib))

    gb = gb_v[pl.ds(0, L)][0]
    iota16 = lax.iota(jnp.int32, L)
    for cp in bias_copies:
        cp.wait()

    def fetch(vec_u, vec_i, w, sems):
        copies = []
        for j in range(G):
            su = vec_u[w * G + j]
            si = vec_i[w * G + j]
            su0 = pl.multiple_of((su >> 7) * 128, 128)
            si0 = pl.multiple_of((si >> 7) * 128, 128)
            copies.append(pltpu.async_copy(
                uemb_hbm.at[:, pl.ds(su0, 128)], ubufs.at[j], sems[0]))
            copies.append(pltpu.async_copy(
                iemb_hbm.at[:, pl.ds(si0, 128)], ibufs.at[j], sems[1]))
        return copies

    def sg_step(sg, carry):
        c = sg // (CHUNK // L)
        off = (sg % (CHUNK // L)) * L
        osl = pl.ds(off, L)
        vec_u = uid_v[c, osl]
        vec_i = iid_v[c, osl]

        abs_rows = off + iota16 + c * CHUNK
        acc = plsc.load_gather(ub_v, [abs_rows])
        acc = acc + plsc.load_gather(ib_v, [abs_rows])
        acc = acc + gb

        for w in range(NWAVE):
            copies = fetch(vec_u, vec_i, w, (sem_u, sem_i))
            for cp in copies:
                cp.wait()
            for j in range(G):
                cu = jnp.full((L,), vec_u[w * G + j] & 127, jnp.int32)
                ci = jnp.full((L,), vec_i[w * G + j] & 127, jnp.int32)
                p = None
                for q in range(D // L):
                    rows = iota16 + q * L
                    uvals = plsc.load_gather(ubufs.at[j], [rows, cu])
                    ivals = plsc.load_gather(ibufs.at[j], [rows, ci])
                    t = uvals * ivals
                    p = t if p is None else p + t
                dot = lax.reduce_sum_p.bind(p, axes=(0,))
                lane = w * G + j
                acc = jnp.where(iota16 == lane, acc + dot, acc)
        out_v[c, osl] = acc
        return carry

    lax.fori_loop(0, NSG, sg_step, 0)

    for c in range(NCHUNK):
        pltpu.sync_copy(out_v.at[c], out_hbm.at[pl.ds(base + c * CHUNK, CHUNK)])


@jax.jit
def _mf_predict(user_ids, item_ids, uemb_t, iemb_t,
                user_bias_w, item_bias_w, global_bias):
    mesh = plsc.VectorSubcoreMesh(core_axis_name="c", subcore_axis_name="s",
                                  num_cores=NC, num_subcores=NS)
    kfn = pl.kernel(
        _body,
        out_type=jax.ShapeDtypeStruct((B,), jnp.float32),
        mesh=mesh,
        scratch_types=[
            pltpu.VMEM((NCHUNK, CHUNK), jnp.int32),
            pltpu.VMEM((NCHUNK, CHUNK), jnp.int32),
            pltpu.VMEM((G, D, 128), jnp.float32),
            pltpu.VMEM((G, D, 128), jnp.float32),
            pltpu.VMEM((BPW,), jnp.float32),
            pltpu.VMEM((BPW,), jnp.float32),
            pltpu.VMEM((L,), jnp.float32),
            pltpu.VMEM((NCHUNK, CHUNK), jnp.float32),
            pltpu.SemaphoreType.DMA,
            pltpu.SemaphoreType.DMA,
            pltpu.SemaphoreType.DMA,
            pltpu.SemaphoreType.DMA,
        ],
        compiler_params=pltpu.CompilerParams(needs_layout_passes=False,
                                             use_tc_tiling_on_sc=True),
    )
    return kfn(user_ids, item_ids, uemb_t, iemb_t,
               user_bias_w, item_bias_w, global_bias)


def kernel(user_ids, item_ids, user_emb, item_emb, user_bias_w, item_bias_w,
           global_bias):
    return _mf_predict(user_ids.astype(jnp.int32), item_ids.astype(jnp.int32),
                       user_emb.T, item_emb.T,
                       user_bias_w.reshape(-1), item_bias_w.reshape(-1),
                       global_bias)


# speedup vs baseline: 2.2521x; 1.1414x over previous
import functools

import jax
import jax.numpy as jnp
from jax import lax
from jax.experimental import pallas as pl
from jax.experimental.pallas import tpu as pltpu
from jax.experimental.pallas import tpu_sc as plsc

B = 16384
D = 64
NC = 2
NS = 16
L = 16
NW = NC * NS
BPW = B // NW
CHUNK = 128
NCHUNK = BPW // CHUNK
G = 2
NWAVE = L // G
NSG = BPW // L


def _body(uid_hbm, iid_hbm, uemb_hbm, iemb_hbm, ubw_hbm, ibw_hbm, gb_hbm,
          out_hbm,
          uid_v, iid_v, ubufs, ibufs, ub_v, ib_v,
          gb_v, out_v, sem_u, sem_i, sem_ub, sem_ib):
    wid = lax.axis_index("s") * NC + lax.axis_index("c")
    base = wid * BPW

    for c in range(NCHUNK):
        src = pl.ds(base + c * CHUNK, CHUNK)
        pltpu.sync_copy(uid_hbm.at[src], uid_v.at[c])
        pltpu.sync_copy(iid_hbm.at[src], iid_v.at[c])
    pltpu.sync_copy(gb_hbm, gb_v.at[pl.ds(0, 1)])

    bias_copies = []
    for c in range(NCHUNK):
        rsl = pl.ds(c * CHUNK, CHUNK)
        bias_copies.append(pltpu.async_copy(ubw_hbm.at[uid_v.at[c]],
                                            ub_v.at[rsl], sem_ub))
        bias_copies.append(pltpu.async_copy(ibw_hbm.at[iid_v.at[c]],
                                            ib_v.at[rsl], sem_ib))

    gb = gb_v[pl.ds(0, L)][0]
    iota16 = lax.iota(jnp.int32, L)
    for cp in bias_copies:
        cp.wait()

    def load_ids(sg):
        c = sg // (CHUNK // L)
        off = (sg % (CHUNK // L)) * L
        return c, pl.ds(off, L)

    def fire(vec_u, vec_i, w, ph):
        for j in range(G):
            su = vec_u[w * G + j]
            si = vec_i[w * G + j]
            su0 = pl.multiple_of((su >> 7) * 128, 128)
            si0 = pl.multiple_of((si >> 7) * 128, 128)
            pltpu.async_copy(uemb_hbm.at[:, pl.ds(su0, 128)],
                             ubufs.at[ph, j], sem_u)
            pltpu.async_copy(iemb_hbm.at[:, pl.ds(si0, 128)],
                             ibufs.at[ph, j], sem_i)

    def drain(ph):
        for j in range(G):
            pltpu.make_async_copy(uemb_hbm.at[:, pl.ds(0, 128)],
                                  ubufs.at[ph, j], sem_u).wait()
            pltpu.make_async_copy(iemb_hbm.at[:, pl.ds(0, 128)],
                                  ibufs.at[ph, j], sem_i).wait()

    c0, osl0 = load_ids(0)
    fire(uid_v[c0, osl0], iid_v[c0, osl0], 0, 0)

    def sg_step(sg, carry):
        c, osl = load_ids(sg)
        vec_u = uid_v[c, osl]
        vec_i = iid_v[c, osl]

        abs_rows = (sg % (CHUNK // L)) * L + iota16 + c * CHUNK
        acc = plsc.load_gather(ub_v, [abs_rows])
        acc = acc + plsc.load_gather(ib_v, [abs_rows])
        acc = acc + gb

        for w in range(NWAVE):
            ph = w % 2
            if w < NWAVE - 1:
                fire(vec_u, vec_i, w + 1, (w + 1) % 2)
            else:
                sgn = jnp.minimum(sg + 1, NSG - 1)
                cn, osln = load_ids(sgn)
                fire(uid_v[cn, osln], iid_v[cn, osln], 0, 0)
            drain(ph)
            for j in range(G):
                lane = w * G + j
                cu = jnp.full((L,), vec_u[lane] & 127, jnp.int32)
                ci = jnp.full((L,), vec_i[lane] & 127, jnp.int32)
                p = None
                for q in range(D // L):
                    rows = iota16 + q * L
                    uvals = plsc.load_gather(ubufs.at[ph, j], [rows, cu])
                    ivals = plsc.load_gather(ibufs.at[ph, j], [rows, ci])
                    t = uvals * ivals
                    p = t if p is None else p + t
                dot = lax.reduce_sum_p.bind(p, axes=(0,))
                acc = jnp.where(iota16 == lane, acc + dot, acc)
        out_v[c, osl] = acc
        return carry

    lax.fori_loop(0, NSG, sg_step, 0)

    drain(0)

    for c in range(NCHUNK):
        pltpu.sync_copy(out_v.at[c], out_hbm.at[pl.ds(base + c * CHUNK, CHUNK)])


@jax.jit
def _mf_predict(user_ids, item_ids, uemb_t, iemb_t,
                user_bias_w, item_bias_w, global_bias):
    mesh = plsc.VectorSubcoreMesh(core_axis_name="c", subcore_axis_name="s",
                                  num_cores=NC, num_subcores=NS)
    kfn = pl.kernel(
        _body,
        out_type=jax.ShapeDtypeStruct((B,), jnp.float32),
        mesh=mesh,
        scratch_types=[
            pltpu.VMEM((NCHUNK, CHUNK), jnp.int32),
            pltpu.VMEM((NCHUNK, CHUNK), jnp.int32),
            pltpu.VMEM((2, G, D, 128), jnp.float32),
            pltpu.VMEM((2, G, D, 128), jnp.float32),
            pltpu.VMEM((BPW,), jnp.float32),
            pltpu.VMEM((BPW,), jnp.float32),
            pltpu.VMEM((L,), jnp.float32),
            pltpu.VMEM((NCHUNK, CHUNK), jnp.float32),
            pltpu.SemaphoreType.DMA,
            pltpu.SemaphoreType.DMA,
            pltpu.SemaphoreType.DMA,
            pltpu.SemaphoreType.DMA,
        ],
        compiler_params=pltpu.CompilerParams(needs_layout_passes=False,
                                             use_tc_tiling_on_sc=True),
    )
    return kfn(user_ids, item_ids, uemb_t, iemb_t,
               user_bias_w, item_bias_w, global_bias)


def kernel(user_ids, item_ids, user_emb, item_emb, user_bias_w, item_bias_w,
           global_bias):
    return _mf_predict(user_ids.astype(jnp.int32), item_ids.astype(jnp.int32),
                       user_emb.T, item_emb.T,
                       user_bias_w.reshape(-1), item_bias_w.reshape(-1),
                       global_bias)


# speedup vs baseline: 2.3876x; 1.0601x over previous
import functools

import jax
import jax.numpy as jnp
from jax import lax
from jax.experimental import pallas as pl
from jax.experimental.pallas import tpu as pltpu
from jax.experimental.pallas import tpu_sc as plsc

B = 16384
D = 64
NC = 2
NS = 16
L = 16
NW = NC * NS
BPW = B // NW
CHUNK = 128
NCHUNK = BPW // CHUNK
NPH = 4
DEPTH = 3
NSG = BPW // L


def _body(uid_hbm, iid_hbm, uemb_hbm, iemb_hbm, ubw_hbm, ibw_hbm, gb_hbm,
          out_hbm,
          uid_v, iid_v, ubufs, ibufs, ub_v, ib_v,
          gb_v, out_v, sem_u, sem_i, sem_ub, sem_ib):
    wid = lax.axis_index("s") * NC + lax.axis_index("c")
    base = wid * BPW

    for c in range(NCHUNK):
        src = pl.ds(base + c * CHUNK, CHUNK)
        pltpu.sync_copy(uid_hbm.at[src], uid_v.at[c])
        pltpu.sync_copy(iid_hbm.at[src], iid_v.at[c])
    pltpu.sync_copy(gb_hbm, gb_v.at[pl.ds(0, 1)])

    bias_copies = []
    for c in range(NCHUNK):
        rsl = pl.ds(c * CHUNK, CHUNK)
        bias_copies.append(pltpu.async_copy(ubw_hbm.at[uid_v.at[c]],
                                            ub_v.at[rsl], sem_ub))
        bias_copies.append(pltpu.async_copy(ibw_hbm.at[iid_v.at[c]],
                                            ib_v.at[rsl], sem_ib))

    gb = gb_v[pl.ds(0, L)][0]
    iota16 = lax.iota(jnp.int32, L)
    for cp in bias_copies:
        cp.wait()

    def load_ids(sg):
        c = sg // (CHUNK // L)
        off = (sg % (CHUNK // L)) * L
        return c, pl.ds(off, L)

    def fire(vec_u, vec_i, lane, ph):
        su = vec_u[lane]
        si = vec_i[lane]
        su0 = pl.multiple_of((su >> 7) * 128, 128)
        si0 = pl.multiple_of((si >> 7) * 128, 128)
        pltpu.async_copy(uemb_hbm.at[:, pl.ds(su0, 128)],
                         ubufs.at[ph], sem_u)
        pltpu.async_copy(iemb_hbm.at[:, pl.ds(si0, 128)],
                         ibufs.at[ph], sem_i)

    def drain(ph):
        pltpu.make_async_copy(uemb_hbm.at[:, pl.ds(0, 128)],
                              ubufs.at[ph], sem_u).wait()
        pltpu.make_async_copy(iemb_hbm.at[:, pl.ds(0, 128)],
                              ibufs.at[ph], sem_i).wait()

    c0, osl0 = load_ids(0)
    for w in range(DEPTH):
        fire(uid_v[c0, osl0], iid_v[c0, osl0], w, w % NPH)

    def sg_step(sg, carry):
        c, osl = load_ids(sg)
        vec_u = uid_v[c, osl]
        vec_i = iid_v[c, osl]
        sgn = jnp.minimum(sg + 1, NSG - 1)
        cn, osln = load_ids(sgn)
        vec_un = uid_v[cn, osln]
        vec_in = iid_v[cn, osln]

        abs_rows = (sg % (CHUNK // L)) * L + iota16 + c * CHUNK
        acc = plsc.load_gather(ub_v, [abs_rows])
        acc = acc + plsc.load_gather(ib_v, [abs_rows])
        acc = acc + gb

        for w in range(L):
            ph = w % NPH
            nxt = w + DEPTH
            if nxt < L:
                fire(vec_u, vec_i, nxt, nxt % NPH)
            else:
                fire(vec_un, vec_in, nxt - L, nxt % NPH)
            drain(ph)
            cu = jnp.full((L,), vec_u[w] & 127, jnp.int32)
            ci = jnp.full((L,), vec_i[w] & 127, jnp.int32)
            p = None
            for q in range(D // L):
                rows = iota16 + q * L
                uvals = plsc.load_gather(ubufs.at[ph], [rows, cu])
                ivals = plsc.load_gather(ibufs.at[ph], [rows, ci])
                t = uvals * ivals
                p = t if p is None else p + t
            dot = lax.reduce_sum_p.bind(p, axes=(0,))
            acc = jnp.where(iota16 == w, acc + dot, acc)
        out_v[c, osl] = acc
        return carry

    lax.fori_loop(0, NSG, sg_step, 0)

    for w in range(DEPTH):
        drain(w % NPH)

    for c in range(NCHUNK):
        pltpu.sync_copy(out_v.at[c], out_hbm.at[pl.ds(base + c * CHUNK, CHUNK)])


@jax.jit
def _mf_predict(user_ids, item_ids, uemb_t, iemb_t,
                user_bias_w, item_bias_w, global_bias):
    mesh = plsc.VectorSubcoreMesh(core_axis_name="c", subcore_axis_name="s",
                                  num_cores=NC, num_subcores=NS)
    kfn = pl.kernel(
        _body,
        out_type=jax.ShapeDtypeStruct((B,), jnp.float32),
        mesh=mesh,
        scratch_types=[
            pltpu.VMEM((NCHUNK, CHUNK), jnp.int32),
            pltpu.VMEM((NCHUNK, CHUNK), jnp.int32),
            pltpu.VMEM((NPH, D, 128), jnp.float32),
            pltpu.VMEM((NPH, D, 128), jnp.float32),
            pltpu.VMEM((BPW,), jnp.float32),
            pltpu.VMEM((BPW,), jnp.float32),
            pltpu.VMEM((L,), jnp.float32),
            pltpu.VMEM((NCHUNK, CHUNK), jnp.float32),
            pltpu.SemaphoreType.DMA,
            pltpu.SemaphoreType.DMA,
            pltpu.SemaphoreType.DMA,
            pltpu.SemaphoreType.DMA,
        ],
        compiler_params=pltpu.CompilerParams(needs_layout_passes=False,
                                             use_tc_tiling_on_sc=True),
    )
    return kfn(user_ids, item_ids, uemb_t, iemb_t,
               user_bias_w, item_bias_w, global_bias)


def kernel(user_ids, item_ids, user_emb, item_emb, user_bias_w, item_bias_w,
           global_bias):
    return _mf_predict(user_ids.astype(jnp.int32), item_ids.astype(jnp.int32),
                       user_emb.T, item_emb.T,
                       user_bias_w.reshape(-1), item_bias_w.reshape(-1),
                       global_bias)


# speedup vs baseline: 2.8079x; 1.1761x over previous
import functools

import jax
import jax.numpy as jnp
from jax import lax
from jax.experimental import pallas as pl
from jax.experimental.pallas import tpu as pltpu
from jax.experimental.pallas import tpu_sc as plsc

B = 16384
D = 64
NC = 2
NS = 16
L = 16
NW = NC * NS
BPW = B // NW
CHUNK = 128
NCHUNK = BPW // CHUNK
NPH = 4
DEPTH = 3
NSG = BPW // L


def _body(uid_hbm, iid_hbm, uemb_hbm, iemb_hbm,
          out_hbm,
          uid_v, iid_v, ubufs, ibufs, out_v, sem_u, sem_i):
    wid = lax.axis_index("s") * NC + lax.axis_index("c")
    base = wid * BPW

    for c in range(NCHUNK):
        src = pl.ds(base + c * CHUNK, CHUNK)
        pltpu.sync_copy(uid_hbm.at[src], uid_v.at[c])
        pltpu.sync_copy(iid_hbm.at[src], iid_v.at[c])

    iota16 = lax.iota(jnp.int32, L)
    zero16 = jnp.zeros((L,), jnp.float32)

    def load_ids(sg):
        c = sg // (CHUNK // L)
        off = (sg % (CHUNK // L)) * L
        return c, pl.ds(off, L)

    def fire(vec_u, vec_i, lane, ph):
        su = vec_u[lane]
        si = vec_i[lane]
        su0 = pl.multiple_of((su >> 7) * 128, 128)
        si0 = pl.multiple_of((si >> 7) * 128, 128)
        pltpu.async_copy(uemb_hbm.at[:, pl.ds(su0, 128)],
                         ubufs.at[ph], sem_u)
        pltpu.async_copy(iemb_hbm.at[:, pl.ds(si0, 128)],
                         ibufs.at[ph], sem_i)

    def drain(ph):
        pltpu.make_async_copy(uemb_hbm.at[:, pl.ds(0, 128)],
                              ubufs.at[ph], sem_u).wait()
        pltpu.make_async_copy(iemb_hbm.at[:, pl.ds(0, 128)],
                              ibufs.at[ph], sem_i).wait()

    c0, osl0 = load_ids(0)
    for w in range(DEPTH):
        fire(uid_v[c0, osl0], iid_v[c0, osl0], w, w % NPH)

    def sg_step(sg, carry):
        c, osl = load_ids(sg)
        vec_u = uid_v[c, osl]
        vec_i = iid_v[c, osl]
        sgn = jnp.minimum(sg + 1, NSG - 1)
        cn, osln = load_ids(sgn)
        vec_un = uid_v[cn, osln]
        vec_in = iid_v[cn, osln]

        acc = zero16

        for w in range(L):
            ph = w % NPH
            nxt = w + DEPTH
            if nxt < L:
                fire(vec_u, vec_i, nxt, nxt % NPH)
            else:
                fire(vec_un, vec_in, nxt - L, nxt % NPH)
            drain(ph)
            cu = jnp.full((L,), vec_u[w] & 127, jnp.int32)
            ci = jnp.full((L,), vec_i[w] & 127, jnp.int32)
            p = None
            for q in range(D // L):
                rows = iota16 + q * L
                uvals = plsc.load_gather(ubufs.at[ph], [rows, cu])
                ivals = plsc.load_gather(ibufs.at[ph], [rows, ci])
                t = uvals * ivals
                p = t if p is None else p + t
            dot = lax.reduce_sum_p.bind(p, axes=(0,))
            acc = jnp.where(iota16 == w, acc + dot, acc)
        out_v[c, osl] = acc
        return carry

    lax.fori_loop(0, NSG, sg_step, 0)

    for w in range(DEPTH):
        drain(w % NPH)

    for c in range(NCHUNK):
        pltpu.sync_copy(out_v.at[c], out_hbm.at[pl.ds(base + c * CHUNK, CHUNK)])


def _bias_body(part_hbm, uid_hbm, iid_hbm, ubw_hbm, ibw_hbm, gb_hbm,
               out_hbm,
               uid_v, iid_v, part_v, ub_v, ib_v, gb_v, out_v,
               sem_ub, sem_ib):
    wid = lax.axis_index("s") * NC + lax.axis_index("c")
    base = wid * BPW

    for c in range(NCHUNK):
        src = pl.ds(base + c * CHUNK, CHUNK)
        pltpu.sync_copy(uid_hbm.at[src], uid_v.at[c])
        pltpu.sync_copy(iid_hbm.at[src], iid_v.at[c])
        pltpu.sync_copy(part_hbm.at[src], part_v.at[c])
    pltpu.sync_copy(gb_hbm, gb_v.at[pl.ds(0, 1)])

    copies = []
    for c in range(NCHUNK):
        rsl = pl.ds(c * CHUNK, CHUNK)
        copies.append(pltpu.async_copy(ubw_hbm.at[uid_v.at[c]],
                                       ub_v.at[rsl], sem_ub))
        copies.append(pltpu.async_copy(ibw_hbm.at[iid_v.at[c]],
                                       ib_v.at[rsl], sem_ib))
    gb = gb_v[pl.ds(0, L)][0]
    for cp in copies:
        cp.wait()

    for c in range(NCHUNK):
        for g in range(CHUNK // L):
            gsl = pl.ds(g * L, L)
            asl = pl.ds(c * CHUNK + g * L, L)
            out_v[c, gsl] = part_v[c, gsl] + ub_v[asl] + ib_v[asl] + gb

    for c in range(NCHUNK):
        pltpu.sync_copy(out_v.at[c], out_hbm.at[pl.ds(base + c * CHUNK, CHUNK)])


@jax.jit
def _mf_predict(user_ids, item_ids, uemb_t, iemb_t,
                user_bias_w, item_bias_w, global_bias):
    mesh = plsc.VectorSubcoreMesh(core_axis_name="c", subcore_axis_name="s",
                                  num_cores=NC, num_subcores=NS)
    kfn = pl.kernel(
        _body,
        out_type=jax.ShapeDtypeStruct((B,), jnp.float32),
        mesh=mesh,
        scratch_types=[
            pltpu.VMEM((NCHUNK, CHUNK), jnp.int32),
            pltpu.VMEM((NCHUNK, CHUNK), jnp.int32),
            pltpu.VMEM((NPH, D, 128), jnp.float32),
            pltpu.VMEM((NPH, D, 128), jnp.float32),
            pltpu.VMEM((NCHUNK, CHUNK), jnp.float32),
            pltpu.SemaphoreType.DMA,
            pltpu.SemaphoreType.DMA,
        ],
        compiler_params=pltpu.CompilerParams(needs_layout_passes=False,
                                             use_tc_tiling_on_sc=True),
    )
    part = kfn(user_ids, item_ids, uemb_t, iemb_t)
    bfn = pl.kernel(
        _bias_body,
        out_type=jax.ShapeDtypeStruct((B,), jnp.float32),
        mesh=mesh,
        scratch_types=[
            pltpu.VMEM((NCHUNK, CHUNK), jnp.int32),
            pltpu.VMEM((NCHUNK, CHUNK), jnp.int32),
            pltpu.VMEM((NCHUNK, CHUNK), jnp.float32),
            pltpu.VMEM((BPW,), jnp.float32),
            pltpu.VMEM((BPW,), jnp.float32),
            pltpu.VMEM((L,), jnp.float32),
            pltpu.VMEM((NCHUNK, CHUNK), jnp.float32),
            pltpu.SemaphoreType.DMA,
            pltpu.SemaphoreType.DMA,
        ],
        compiler_params=pltpu.CompilerParams(needs_layout_passes=False),
    )
    return bfn(part, user_ids, item_ids,
               user_bias_w, item_bias_w, global_bias)


def kernel(user_ids, item_ids, user_emb, item_emb, user_bias_w, item_bias_w,
           global_bias):
    return _mf_predict(user_ids.astype(jnp.int32), item_ids.astype(jnp.int32),
                       user_emb.T, item_emb.T,
                       user_bias_w.reshape(-1), item_bias_w.reshape(-1),
                       global_bias)
